# baseline (speedup 1.0000x reference)
import functools

import jax
import jax.numpy as jnp
from jax import lax
from jax.experimental import pallas as pl
from jax.experimental.pallas import tpu as pltpu
from jax.experimental.pallas import tpu_sc as plsc

N = 10000
NPAD = 10112
E = 320000
L = 16
NC = 2
NS = 16
NW = NC * NS
RNG = NPAD // NW
E2 = E + N
SCH = 2048
NCH = -(-E2 // SCH)
E2P = NCH * SCH
CAP = 12800
CAPP = CAP + 32
BIGDST = 0x3FFFFFFF

_MESH = plsc.VectorSubcoreMesh(
    core_axis_name="c", subcore_axis_name="s", num_cores=NC, num_subcores=NS)
_SC_PARAMS = pltpu.CompilerParams(
    needs_layout_passes=False, use_tc_tiling_on_sc=False)


def _wid():
    return lax.axis_index("c") * NS + lax.axis_index("s")


@functools.partial(
    pl.kernel,
    out_type=(
        jax.ShapeDtypeStruct((NW, CAP), jnp.int32),
        jax.ShapeDtypeStruct((NW, CAP), jnp.int32),
        jax.ShapeDtypeStruct((NW, CAP), jnp.int32),
        jax.ShapeDtypeStruct((NW, L), jnp.int32),
        jax.ShapeDtypeStruct((NW, RNG), jnp.float32),
    ),
    mesh=_MESH,
    compiler_params=_SC_PARAMS,
    scratch_types=[
        pltpu.VMEM((3, SCH), jnp.int32),
        pltpu.VMEM((3, SCH), jnp.int32),
        pltpu.VMEM((CAPP,), jnp.int32),
        pltpu.VMEM((CAPP,), jnp.int32),
        pltpu.VMEM((CAPP,), jnp.int32),
        pltpu.VMEM((RNG + 20,), jnp.float32),
        pltpu.VMEM((L,), jnp.int32),
        pltpu.SemaphoreType.DMA,
        pltpu.SemaphoreType.DMA,
    ],
)
def _scan_kernel(stream_hbm,
                 srcl_hbm, ewl_hbm, dstl_hbm, cnt_hbm, deg_hbm,
                 buf0, buf1, srcl, ewl, dstl, degv, cbuf, sem0, sem1):
    wid = _wid()
    lo = wid * RNG
    hi = lo + RNG

    zi = jnp.zeros((L,), jnp.int32)
    zf = jnp.zeros((L,), jnp.float32)
    lo16 = jnp.full((L,), lo, jnp.int32)
    for i in range(CAPP // L):
        srcl[pl.ds(i * L, L)] = zi
        ewl[pl.ds(i * L, L)] = zi
        dstl[pl.ds(i * L, L)] = lo16
    for i in range((RNG + 20) // L + 1):
        degv[pl.ds(min(i * L, RNG + 20 - L), L)] = zf

    def start_copy(idx, buf, sem):
        cidx = jnp.minimum(idx, NCH - 1)
        pltpu.async_copy(stream_hbm.at[cidx], buf, sem)

    def process_chunk(buf, pos):
        def vreg_body(i, pos):
            s16 = buf[0, pl.ds(i * L, L)]
            d16 = buf[1, pl.ds(i * L, L)]
            e16 = buf[2, pl.ds(i * L, L)]
            m = (d16 >= lo) & (d16 < hi)
            plsc.store_compressed(srcl.at[pl.ds(pos, L)], s16, mask=m)
            plsc.store_compressed(ewl.at[pl.ds(pos, L)], e16, mask=m)
            plsc.store_compressed(dstl.at[pl.ds(pos, L)], d16, mask=m)
            pc = plsc.all_reduce_population_count(m)
            return pos + pc[0]

        return lax.fori_loop(0, SCH // L, vreg_body, pos)

    hbm_dummy = stream_hbm.at[0]
    start_copy(jnp.int32(0), buf0, sem0)

    def pair_body(j, pos):
        pltpu.make_async_copy(hbm_dummy, buf0, sem0).wait()
        start_copy(2 * j + 1, buf1, sem1)
        pos = process_chunk(buf0, pos)
        pltpu.make_async_copy(hbm_dummy, buf1, sem1).wait()
        start_copy(2 * j + 2, buf0, sem0)
        pos = process_chunk(buf1, pos)
        return pos

    cnt = lax.fori_loop(0, NCH // 2, pair_body, jnp.int32(0))
    pltpu.make_async_copy(hbm_dummy, buf0, sem0).wait()

    onehot0 = lax.iota(jnp.int32, L) == 0

    def deg_body(g, carry):
        d16 = dstl[pl.ds(g * L, L)] - lo
        e16 = plsc.bitcast(ewl[pl.ds(g * L, L)], jnp.float32)
        for i in range(L):
            idx = d16[i]
            vec = degv[pl.ds(idx, L)]
            degv[pl.ds(idx, L)] = vec + jnp.where(onehot0, e16[i], 0.0)
        return carry

    lax.fori_loop(0, (cnt + L - 1) // L, deg_body, 0)

    pltpu.sync_copy(degv.at[pl.ds(0, RNG)], deg_hbm.at[wid])
    pltpu.sync_copy(srcl.at[pl.ds(0, CAP)], srcl_hbm.at[wid])
    pltpu.sync_copy(ewl.at[pl.ds(0, CAP)], ewl_hbm.at[wid])
    pltpu.sync_copy(dstl.at[pl.ds(0, CAP)], dstl_hbm.at[wid])
    cbuf[...] = jnp.full((L,), cnt, jnp.int32)
    pltpu.sync_copy(cbuf, cnt_hbm.at[wid])


@functools.partial(
    pl.kernel,
    out_type=jax.ShapeDtypeStruct((NW, CAP), jnp.float32),
    mesh=_MESH,
    compiler_params=_SC_PARAMS,
    scratch_types=[
        pltpu.VMEM((CAP,), jnp.int32),
        pltpu.VMEM((CAP,), jnp.int32),
        pltpu.VMEM((CAP,), jnp.float32),
        pltpu.VMEM((NPAD,), jnp.float32),
        pltpu.VMEM((CAP,), jnp.float32),
    ],
)
def _norm_kernel(srcl_hbm, dstl_hbm, ewl_hbm, dinv_hbm, out_hbm,
                 srcv, dstv, ewv, dv, nbuf):
    wid = _wid()
    pltpu.sync_copy(srcl_hbm.at[wid], srcv)
    pltpu.sync_copy(dstl_hbm.at[wid], dstv)
    pltpu.sync_copy(ewl_hbm.at[wid], ewv)
    pltpu.sync_copy(dinv_hbm, dv)

    def body(g, carry):
        s16 = srcv[pl.ds(g * L, L)]
        d16 = dstv[pl.ds(g * L, L)]
        e16 = ewv[pl.ds(g * L, L)]
        n16 = (plsc.load_gather(dv, [s16]) * e16) * plsc.load_gather(dv, [d16])
        nbuf[pl.ds(g * L, L)] = n16
        return carry

    lax.fori_loop(0, CAP // L, body, 0)
    pltpu.sync_copy(nbuf, out_hbm.at[wid])


def _make_prop(w):
    G = w // L

    @functools.partial(
        pl.kernel,
        out_type=jax.ShapeDtypeStruct((NW, G, RNG, L), jnp.float32),
        mesh=_MESH,
        compiler_params=_SC_PARAMS,
        scratch_types=[
            pltpu.VMEM((CAP,), jnp.int32),
            pltpu.VMEM((CAP,), jnp.int32),
            pltpu.VMEM((CAP,), jnp.float32),
            pltpu.VMEM((128, w), jnp.float32),
            pltpu.VMEM((128, w), jnp.float32),
            pltpu.VMEM((L,), jnp.int32),
            pltpu.SemaphoreType.DMA,
            pltpu.SemaphoreType.DMA,
        ] + [pltpu.VMEM((RNG, L), jnp.float32) for _ in range(G)],
    )
    def _prop(g_hbm, srcl_hbm, dstl_hbm, nrml_hbm, cnt_hbm, zeros_hbm, out_hbm,
              srcv, dstv, nrmv, rows0, rows1, cnts, sem0, sem1, *accs):
        wid = _wid()
        lo = wid * RNG
        pltpu.sync_copy(srcl_hbm.at[wid], srcv)
        pltpu.sync_copy(dstl_hbm.at[wid], dstv)
        pltpu.sync_copy(nrml_hbm.at[wid], nrmv)
        pltpu.sync_copy(cnt_hbm.at[wid], cnts)
        for k in range(G):
            pltpu.sync_copy(zeros_hbm, accs[k])
        cnt = cnts[pl.ds(0, L)][0]

        max_chunk = CAP // 128 - 1

        def start_gather(idx, buf, sem):
            cidx = jnp.minimum(idx, max_chunk)
            pltpu.async_copy(g_hbm.at[srcv.at[pl.ds(cidx * 128, 128)]], buf, sem)

        def process(base, rows):
            def group_body(g2, carry):
                off = base * 128 + g2 * L
                d16 = dstv[pl.ds(off, L)] - lo
                n16 = nrmv[pl.ds(off, L)]
                for i in range(L):
                    drel = d16[i]
                    nb = jnp.full((L,), n16[i], jnp.float32)
                    r = g2 * L + i
                    for k in range(G):
                        accs[k][drel] = (accs[k][drel]
                                         + rows[r, pl.ds(k * L, L)] * nb)
                return carry

            lax.fori_loop(0, 128 // L, group_body, 0, unroll=2)

        start_gather(jnp.int32(0), rows0, sem0)
        npairs = (cnt + 255) // 256

        hbm_dummy = g_hbm.at[pl.ds(0, 128)]

        def pair_body(j, carry):
            pltpu.make_async_copy(hbm_dummy, rows0, sem0).wait()
            start_gather(2 * j + 1, rows1, sem1)
            process(2 * j, rows0)
            pltpu.make_async_copy(hbm_dummy, rows1, sem1).wait()
            start_gather(2 * j + 2, rows0, sem0)
            process(2 * j + 1, rows1)
            return carry

        lax.fori_loop(0, npairs, pair_body, 0)
        pltpu.make_async_copy(hbm_dummy, rows0, sem0).wait()

        for k in range(G):
            pltpu.sync_copy(accs[k], out_hbm.at[wid, k])

    return _prop


_PROP = {w: _make_prop(w) for w in (128, 64, 32, 16)}


_EPS = 1e-5


def _bn_relu(t, gamma, beta):
    mu = jnp.mean(t, axis=0)
    var = jnp.var(t, axis=0)
    return jax.nn.relu(gamma * (t - mu) / jnp.sqrt(var + _EPS) + beta)


def _dot(a, b):
    return jnp.dot(a, b, preferred_element_type=jnp.float32)


def _tc0_body(deg_ref, x_ref, w1_ref, dinv_ref, h1a_ref, h1b_ref):
    deg = deg_ref[...]
    dinv_ref[...] = jnp.where(deg > 0, lax.rsqrt(deg), 0.0)
    h1 = _dot(x_ref[...], w1_ref[...])
    h1a_ref[...] = h1[:, :128]
    h1b_ref[...] = h1[:, 128:]


def _tc_first_body(pa_ref, pb_ref, b_ref, g_ref, be_ref, w2_ref, out_ref):
    t = jnp.concatenate([pa_ref[...][:N], pb_ref[...][:N]], axis=1) + b_ref[...]
    h = _bn_relu(t, g_ref[...], be_ref[...])
    out_ref[...] = _dot(h, w2_ref[...])


def _tc_mid_body(p_ref, b_ref, g_ref, be_ref, wn_ref, out_ref):
    t = p_ref[...][:N] + b_ref[...]
    h = _bn_relu(t, g_ref[...], be_ref[...])
    out_ref[...] = _dot(h, wn_ref[...])


def _tc_final_body(p_ref, b_ref, g_ref, be_ref,
                   m1_ref, c1_ref, mg1_ref, mb1_ref,
                   m2_ref, c2_ref, mg2_ref, mb2_ref,
                   m3_ref, c3_ref, mg3_ref, mb3_ref,
                   out_ref):
    t = p_ref[...][:N] + b_ref[...]
    h = _bn_relu(t, g_ref[...], be_ref[...])
    z = _bn_relu(_dot(h, m1_ref[...]) + c1_ref[...], mg1_ref[...], mb1_ref[...])
    z = _bn_relu(_dot(z, m2_ref[...]) + c2_ref[...], mg2_ref[...], mb2_ref[...])
    out_ref[...] = _bn_relu(_dot(z, m3_ref[...]) + c3_ref[...],
                            mg3_ref[...], mb3_ref[...])


def _tc(body, out_shapes, *args):
    if isinstance(out_shapes, tuple) and isinstance(out_shapes[0], tuple):
        out_shape = tuple(jax.ShapeDtypeStruct(s, jnp.float32) for s in out_shapes)
    else:
        out_shape = jax.ShapeDtypeStruct(out_shapes, jnp.float32)
    return pl.pallas_call(body, out_shape=out_shape)(*args)


def kernel(x, edge_index, edge_attr, gcn_params, mlp_params):
    i32 = jnp.int32
    f32 = jnp.float32
    src = edge_index[0].astype(i32)
    dst = edge_index[1].astype(i32)
    ew = edge_attr.astype(f32)
    loop = jnp.arange(N, dtype=i32)
    padn = E2P - E2

    src_all = jnp.concatenate([src, loop, jnp.zeros((padn,), i32)]).reshape(NCH, SCH)
    dst_all = jnp.concatenate(
        [dst, loop, jnp.full((padn,), BIGDST, i32)]).reshape(NCH, SCH)
    ew_bits = lax.bitcast_convert_type(
        jnp.concatenate([ew, jnp.ones((N,), f32), jnp.zeros((padn,), f32)]),
        i32).reshape(NCH, SCH)
    stream = jnp.stack([src_all, dst_all, ew_bits], axis=1)

    srcl, ewl_bits, dstl, cnts, deg = _scan_kernel(stream)
    ewl = lax.bitcast_convert_type(ewl_bits, f32)

    (w1, b1, g1, be1), (w2, b2, g2, be2), (w3, b3, g3, be3), \
        (w4, b4, g4, be4), (w5, b5, g5, be5) = gcn_params

    dinv2, h1a, h1b = _tc(
        _tc0_body, (((NW, RNG), (N, 128), (N, 128))), deg, x, w1)
    nrml = _norm_kernel(srcl, dstl, ewl, dinv2.reshape(NPAD))

    zz = jnp.zeros((RNG, L), f32)

    def _asm(p, w):
        return p.transpose(0, 2, 1, 3).reshape(NPAD, w)

    pa = _asm(_PROP[128](h1a, srcl, dstl, nrml, cnts, zz), 128)
    pb = _asm(_PROP[128](h1b, srcl, dstl, nrml, cnts, zz), 128)
    gg2 = _tc(_tc_first_body, (N, 128), pa, pb, b1, g1, be1, w2)
    p2 = _asm(_PROP[128](gg2, srcl, dstl, nrml, cnts, zz), 128)
    gg3 = _tc(_tc_mid_body, (N, 64), p2, b2, g2, be2, w3)
    p3 = _asm(_PROP[64](gg3, srcl, dstl, nrml, cnts, zz), 64)
    gg4 = _tc(_tc_mid_body, (N, 32), p3, b3, g3, be3, w4)
    p4 = _asm(_PROP[32](gg4, srcl, dstl, nrml, cnts, zz), 32)
    gg5 = _tc(_tc_mid_body, (N, 16), p4, b4, g4, be4, w5)
    p5 = _asm(_PROP[16](gg5, srcl, dstl, nrml, cnts, zz), 16)

    (m1, c1, mg1, mb1), (m2, c2, mg2, mb2), (m3, c3, mg3, mb3) = mlp_params
    out = _tc(_tc_final_body, (N, 40), p5, b5, g5, be5,
              m1, c1, mg1, mb1, m2, c2, mg2, mb2, m3, c3, mg3, mb3)
    return out


# speedup vs baseline: 1.1525x; 1.1525x over previous
import functools

import jax
import jax.numpy as jnp
from jax import lax
from jax.experimental import pallas as pl
from jax.experimental.pallas import tpu as pltpu
from jax.experimental.pallas import tpu_sc as plsc

N = 10000
NPAD = 10112
E = 320000
L = 16
NC = 2
NS = 16
NW = NC * NS
RNG = NPAD // NW
E2 = E + N
SCH = 2048
NCH = -(-E2 // SCH)
E2P = NCH * SCH
CAP = 12800
CAPP = CAP + 32
BIGDST = 0x3FFFFFFF

_MESH = plsc.VectorSubcoreMesh(
    core_axis_name="c", subcore_axis_name="s", num_cores=NC, num_subcores=NS)
_SC_PARAMS = pltpu.CompilerParams(
    needs_layout_passes=False, use_tc_tiling_on_sc=False)


def _wid():
    return lax.axis_index("c") * NS + lax.axis_index("s")


@functools.partial(
    pl.kernel,
    out_type=(
        jax.ShapeDtypeStruct((NW, CAP), jnp.int32),
        jax.ShapeDtypeStruct((NW, CAP), jnp.int32),
        jax.ShapeDtypeStruct((NW, CAP), jnp.int32),
        jax.ShapeDtypeStruct((NW, L), jnp.int32),
        jax.ShapeDtypeStruct((NW, RNG), jnp.float32),
    ),
    mesh=_MESH,
    compiler_params=_SC_PARAMS,
    scratch_types=[
        pltpu.VMEM((3, SCH), jnp.int32),
        pltpu.VMEM((3, SCH), jnp.int32),
        pltpu.VMEM((CAPP,), jnp.int32),
        pltpu.VMEM((CAPP,), jnp.int32),
        pltpu.VMEM((CAPP,), jnp.int32),
        pltpu.VMEM((RNG + 20,), jnp.float32),
        pltpu.VMEM((L,), jnp.int32),
        pltpu.SemaphoreType.DMA,
        pltpu.SemaphoreType.DMA,
    ],
)
def _scan_kernel(stream_hbm,
                 srcl_hbm, ewl_hbm, dstl_hbm, cnt_hbm, deg_hbm,
                 buf0, buf1, srcl, ewl, dstl, degv, cbuf, sem0, sem1):
    wid = _wid()
    lo = wid * RNG
    hi = lo + RNG

    zi = jnp.zeros((L,), jnp.int32)
    zf = jnp.zeros((L,), jnp.float32)
    lo16 = jnp.full((L,), lo, jnp.int32)
    for i in range(CAPP // L):
        srcl[pl.ds(i * L, L)] = zi
        ewl[pl.ds(i * L, L)] = zi
        dstl[pl.ds(i * L, L)] = lo16
    for i in range((RNG + 20) // L + 1):
        degv[pl.ds(min(i * L, RNG + 20 - L), L)] = zf

    def start_copy(idx, buf, sem):
        cidx = jnp.minimum(idx, NCH - 1)
        pltpu.async_copy(stream_hbm.at[cidx], buf, sem)

    def process_chunk(buf, pos):
        def vreg_body(i, pos):
            s16 = buf[0, pl.ds(i * L, L)]
            d16 = buf[1, pl.ds(i * L, L)]
            e16 = buf[2, pl.ds(i * L, L)]
            m = (d16 >= lo) & (d16 < hi)
            plsc.store_compressed(srcl.at[pl.ds(pos, L)], s16, mask=m)
            plsc.store_compressed(ewl.at[pl.ds(pos, L)], e16, mask=m)
            plsc.store_compressed(dstl.at[pl.ds(pos, L)], d16, mask=m)
            pc = plsc.all_reduce_population_count(m)
            return pos + pc[0]

        return lax.fori_loop(0, SCH // L, vreg_body, pos)

    hbm_dummy = stream_hbm.at[0]
    start_copy(jnp.int32(0), buf0, sem0)

    def pair_body(j, pos):
        pltpu.make_async_copy(hbm_dummy, buf0, sem0).wait()
        start_copy(2 * j + 1, buf1, sem1)
        pos = process_chunk(buf0, pos)
        pltpu.make_async_copy(hbm_dummy, buf1, sem1).wait()
        start_copy(2 * j + 2, buf0, sem0)
        pos = process_chunk(buf1, pos)
        return pos

    cnt = lax.fori_loop(0, NCH // 2, pair_body, jnp.int32(0))
    pltpu.make_async_copy(hbm_dummy, buf0, sem0).wait()

    onehot0 = lax.iota(jnp.int32, L) == 0

    def deg_body(g, carry):
        d16 = dstl[pl.ds(g * L, L)] - lo
        e16 = plsc.bitcast(ewl[pl.ds(g * L, L)], jnp.float32)
        for i in range(L):
            idx = d16[i]
            vec = degv[pl.ds(idx, L)]
            degv[pl.ds(idx, L)] = vec + jnp.where(onehot0, e16[i], 0.0)
        return carry

    lax.fori_loop(0, (cnt + L - 1) // L, deg_body, 0)

    pltpu.sync_copy(degv.at[pl.ds(0, RNG)], deg_hbm.at[wid])
    pltpu.sync_copy(srcl.at[pl.ds(0, CAP)], srcl_hbm.at[wid])
    pltpu.sync_copy(ewl.at[pl.ds(0, CAP)], ewl_hbm.at[wid])
    pltpu.sync_copy(dstl.at[pl.ds(0, CAP)], dstl_hbm.at[wid])
    cbuf[...] = jnp.full((L,), cnt, jnp.int32)
    pltpu.sync_copy(cbuf, cnt_hbm.at[wid])


@functools.partial(
    pl.kernel,
    out_type=jax.ShapeDtypeStruct((NW, CAP), jnp.float32),
    mesh=_MESH,
    compiler_params=_SC_PARAMS,
    scratch_types=[
        pltpu.VMEM((CAP,), jnp.int32),
        pltpu.VMEM((CAP,), jnp.int32),
        pltpu.VMEM((CAP,), jnp.float32),
        pltpu.VMEM((NPAD,), jnp.float32),
        pltpu.VMEM((CAP,), jnp.float32),
    ],
)
def _norm_kernel(srcl_hbm, dstl_hbm, ewl_hbm, dinv_hbm, out_hbm,
                 srcv, dstv, ewv, dv, nbuf):
    wid = _wid()
    pltpu.sync_copy(srcl_hbm.at[wid], srcv)
    pltpu.sync_copy(dstl_hbm.at[wid], dstv)
    pltpu.sync_copy(ewl_hbm.at[wid], ewv)
    pltpu.sync_copy(dinv_hbm, dv)

    def body(g, carry):
        s16 = srcv[pl.ds(g * L, L)]
        d16 = dstv[pl.ds(g * L, L)]
        e16 = ewv[pl.ds(g * L, L)]
        n16 = (plsc.load_gather(dv, [s16]) * e16) * plsc.load_gather(dv, [d16])
        nbuf[pl.ds(g * L, L)] = n16
        return carry

    lax.fori_loop(0, CAP // L, body, 0)
    pltpu.sync_copy(nbuf, out_hbm.at[wid])


def _make_prop(w):
    G = w // L

    @functools.partial(
        pl.kernel,
        out_type=jax.ShapeDtypeStruct((NW, G, RNG, L), jnp.float32),
        mesh=_MESH,
        compiler_params=_SC_PARAMS,
        scratch_types=[
            pltpu.VMEM((CAP,), jnp.int32),
            pltpu.VMEM((CAP,), jnp.int32),
            pltpu.VMEM((CAP,), jnp.float32),
            pltpu.VMEM((128, w), jnp.float32),
            pltpu.VMEM((128, w), jnp.float32),
            pltpu.VMEM((L,), jnp.int32),
            pltpu.SemaphoreType.DMA,
            pltpu.SemaphoreType.DMA,
        ] + [pltpu.VMEM((RNG, L), jnp.float32) for _ in range(G)],
    )
    def _prop(g_hbm, srcl_hbm, dstl_hbm, nrml_hbm, cnt_hbm, zeros_hbm, out_hbm,
              srcv, dstv, nrmv, rows0, rows1, cnts, sem0, sem1, *accs):
        wid = _wid()
        lo = wid * RNG
        pltpu.sync_copy(srcl_hbm.at[wid], srcv)
        pltpu.sync_copy(dstl_hbm.at[wid], dstv)
        pltpu.sync_copy(nrml_hbm.at[wid], nrmv)
        pltpu.sync_copy(cnt_hbm.at[wid], cnts)
        for k in range(G):
            pltpu.sync_copy(zeros_hbm, accs[k])
        cnt = cnts[pl.ds(0, L)][0]

        max_chunk = CAP // 128 - 1

        def start_gather(idx, buf, sem):
            cidx = jnp.minimum(idx, max_chunk)
            pltpu.async_copy(g_hbm.at[srcv.at[pl.ds(cidx * 128, 128)]], buf, sem)

        def process(base, rows):
            def group_body(g2, carry):
                off = base * 128 + g2 * L
                d16 = dstv[pl.ds(off, L)] - lo
                n16 = nrmv[pl.ds(off, L)]
                for i in range(L):
                    drel = d16[i]
                    nb = jnp.full((L,), n16[i], jnp.float32)
                    r = g2 * L + i
                    for k in range(G):
                        accs[k][drel] = (accs[k][drel]
                                         + rows[r, pl.ds(k * L, L)] * nb)
                return carry

            lax.fori_loop(0, 128 // L, group_body, 0)

        start_gather(jnp.int32(0), rows0, sem0)
        npairs = (cnt + 255) // 256

        hbm_dummy = g_hbm.at[pl.ds(0, 128)]

        def pair_body(j, carry):
            pltpu.make_async_copy(hbm_dummy, rows0, sem0).wait()
            start_gather(2 * j + 1, rows1, sem1)
            process(2 * j, rows0)
            pltpu.make_async_copy(hbm_dummy, rows1, sem1).wait()
            start_gather(2 * j + 2, rows0, sem0)
            process(2 * j + 1, rows1)
            return carry

        lax.fori_loop(0, npairs, pair_body, 0)
        pltpu.make_async_copy(hbm_dummy, rows0, sem0).wait()

        for k in range(G):
            pltpu.sync_copy(accs[k], out_hbm.at[wid, k])

    return _prop


_PROP = {w: _make_prop(w) for w in (128, 64, 32, 16)}


_EPS = 1e-5


def _bn_relu(t, gamma, beta):
    mu = jnp.mean(t, axis=0)
    var = jnp.var(t, axis=0)
    return jax.nn.relu(gamma * (t - mu) / jnp.sqrt(var + _EPS) + beta)


def _dot(a, b):
    return jnp.dot(a, b, preferred_element_type=jnp.float32)


def _tc0_body(deg_ref, x_ref, w1_ref, dinv_ref, h1a_ref, h1b_ref):
    deg = deg_ref[...]
    dinv_ref[...] = jnp.where(deg > 0, lax.rsqrt(deg), 0.0)
    h1 = _dot(x_ref[...], w1_ref[...])
    h1a_ref[...] = h1[:, :128]
    h1b_ref[...] = h1[:, 128:]


def _tc_first_body(pa_ref, pb_ref, b_ref, g_ref, be_ref, w2_ref, out_ref):
    t = jnp.concatenate([pa_ref[...][:N], pb_ref[...][:N]], axis=1) + b_ref[...]
    h = _bn_relu(t, g_ref[...], be_ref[...])
    out_ref[...] = _dot(h, w2_ref[...])


def _tc_mid_body(p_ref, b_ref, g_ref, be_ref, wn_ref, out_ref):
    t = p_ref[...][:N] + b_ref[...]
    h = _bn_relu(t, g_ref[...], be_ref[...])
    out_ref[...] = _dot(h, wn_ref[...])


def _tc_final_body(p_ref, b_ref, g_ref, be_ref,
                   m1_ref, c1_ref, mg1_ref, mb1_ref,
                   m2_ref, c2_ref, mg2_ref, mb2_ref,
                   m3_ref, c3_ref, mg3_ref, mb3_ref,
                   out_ref):
    t = p_ref[...][:N] + b_ref[...]
    h = _bn_relu(t, g_ref[...], be_ref[...])
    z = _bn_relu(_dot(h, m1_ref[...]) + c1_ref[...], mg1_ref[...], mb1_ref[...])
    z = _bn_relu(_dot(z, m2_ref[...]) + c2_ref[...], mg2_ref[...], mb2_ref[...])
    out_ref[...] = _bn_relu(_dot(z, m3_ref[...]) + c3_ref[...],
                            mg3_ref[...], mb3_ref[...])


def _tc(body, out_shapes, *args):
    if isinstance(out_shapes, tuple) and isinstance(out_shapes[0], tuple):
        out_shape = tuple(jax.ShapeDtypeStruct(s, jnp.float32) for s in out_shapes)
    else:
        out_shape = jax.ShapeDtypeStruct(out_shapes, jnp.float32)
    return pl.pallas_call(body, out_shape=out_shape)(*args)


def kernel(x, edge_index, edge_attr, gcn_params, mlp_params):
    i32 = jnp.int32
    f32 = jnp.float32
    src = edge_index[0].astype(i32)
    dst = edge_index[1].astype(i32)
    ew = edge_attr.astype(f32)
    loop = jnp.arange(N, dtype=i32)
    padn = E2P - E2

    src_all = jnp.concatenate([src, loop, jnp.zeros((padn,), i32)]).reshape(NCH, SCH)
    dst_all = jnp.concatenate(
        [dst, loop, jnp.full((padn,), BIGDST, i32)]).reshape(NCH, SCH)
    ew_bits = lax.bitcast_convert_type(
        jnp.concatenate([ew, jnp.ones((N,), f32), jnp.zeros((padn,), f32)]),
        i32).reshape(NCH, SCH)
    stream = jnp.stack([src_all, dst_all, ew_bits], axis=1)

    srcl, ewl_bits, dstl, cnts, deg = _scan_kernel(stream)
    ewl = lax.bitcast_convert_type(ewl_bits, f32)

    (w1, b1, g1, be1), (w2, b2, g2, be2), (w3, b3, g3, be3), \
        (w4, b4, g4, be4), (w5, b5, g5, be5) = gcn_params

    dinv2, h1a, h1b = _tc(
        _tc0_body, (((NW, RNG), (N, 128), (N, 128))), deg, x, w1)
    nrml = _norm_kernel(srcl, dstl, ewl, dinv2.reshape(NPAD))

    zz = jnp.zeros((RNG, L), f32)

    def _asm(p, w):
        return p.transpose(0, 2, 1, 3).reshape(NPAD, w)

    pa = _asm(_PROP[128](h1a, srcl, dstl, nrml, cnts, zz), 128)
    pb = _asm(_PROP[128](h1b, srcl, dstl, nrml, cnts, zz), 128)
    gg2 = _tc(_tc_first_body, (N, 128), pa, pb, b1, g1, be1, w2)
    p2 = _asm(_PROP[128](gg2, srcl, dstl, nrml, cnts, zz), 128)
    gg3 = _tc(_tc_mid_body, (N, 64), p2, b2, g2, be2, w3)
    p3 = _asm(_PROP[64](gg3, srcl, dstl, nrml, cnts, zz), 64)
    gg4 = _tc(_tc_mid_body, (N, 32), p3, b3, g3, be3, w4)
    p4 = _asm(_PROP[32](gg4, srcl, dstl, nrml, cnts, zz), 32)
    gg5 = _tc(_tc_mid_body, (N, 16), p4, b4, g4, be4, w5)
    p5 = _asm(_PROP[16](gg5, srcl, dstl, nrml, cnts, zz), 16)

    (m1, c1, mg1, mb1), (m2, c2, mg2, mb2), (m3, c3, mg3, mb3) = mlp_params
    out = _tc(_tc_final_body, (N, 40), p5, b5, g5, be5,
              m1, c1, mg1, mb1, m2, c2, mg2, mb2, m3, c3, mg3, mb3)
    return out
